# baseline (speedup 1.0000x reference)
import functools

import jax
import jax.numpy as jnp
from jax import lax
from jax.experimental import pallas as pl
from jax.experimental.pallas import tpu as pltpu
from jax.experimental.pallas import tpu_sc as plsc

D = 128
CHUNK = 128
NBUF = 5
AHEAD = 2


@functools.partial(jax.jit, static_argnames=("n_chunks",))
def _sc_gather(table, idx, n_chunks):
    nw = idx.shape[0]
    b_per_w = n_chunks * CHUNK
    n_outer = n_chunks // NBUF
    assert n_outer * NBUF == n_chunks
    mesh = plsc.VectorSubcoreMesh(core_axis_name="c", subcore_axis_name="s")

    @functools.partial(
        pl.kernel,
        mesh=mesh,
        out_type=jax.ShapeDtypeStruct((nw * b_per_w, D), jnp.float32),
        scratch_types=(
            [pltpu.VMEM((n_chunks, CHUNK), jnp.int32),
             pltpu.VMEM((NBUF, CHUNK, D), jnp.float32)]
            + [pltpu.SemaphoreType.DMA] * (2 * NBUF)
        ),
    )
    def k(table_hbm, idx_hbm, out_hbm, idx_v, rows_v, *sems):
        gsem = sems[:NBUF]
        osem = sems[NBUF:]
        wid = lax.axis_index("s") * 2 + lax.axis_index("c")
        base = wid * b_per_w
        pltpu.sync_copy(idx_hbm.at[wid], idx_v)

        def fire(h, slot):
            pltpu.async_copy(
                table_hbm.at[idx_v.at[h]], rows_v.at[slot], gsem[slot])

        def wait_gather(g, slot):
            pltpu.make_async_copy(
                table_hbm.at[idx_v.at[g]], rows_v.at[slot], gsem[slot]).wait()

        def out_copy(slot, g):
            return pltpu.make_async_copy(
                rows_v.at[slot],
                out_hbm.at[pl.ds(base + g * CHUNK, CHUNK)],
                osem[slot])

        for h in range(AHEAD):
            fire(h, h)

        def body(t, carry):
            for b in range(NBUF):
                g = t * NBUF + b
                sh = (b + AHEAD) % NBUF
                h = g + AHEAD

                @pl.when(h < n_chunks)
                def _():
                    @pl.when(h >= NBUF)
                    def _():
                        out_copy(sh, 0).wait()
                    fire(h, sh)

                wait_gather(g, b)
                out_copy(b, g).start()
            return carry

        lax.fori_loop(0, n_outer, body, 0)

        for b in range(NBUF):
            out_copy(b, 0).wait()

    return k(table, idx)


def kernel(indices, embed_weight):
    bsz, seq = indices.shape
    total = bsz * seq
    nw = 32
    n_chunks = total // (nw * CHUNK)
    idx = indices.reshape(nw, n_chunks, CHUNK).astype(jnp.int32)
    out = _sc_gather(embed_weight, idx, n_chunks)
    return out.reshape(bsz, seq, D)


# speedup vs baseline: 1.7037x; 1.7037x over previous
import functools

import jax
import jax.numpy as jnp
from jax import lax
from jax.experimental import pallas as pl
from jax.experimental.pallas import tpu as pltpu
from jax.experimental.pallas import tpu_sc as plsc

D = 128
CHUNK = 128
NBUF = 5
AHEAD = 2


@functools.partial(jax.jit, static_argnames=("n_chunks",))
def _sc_gather(table, idx, n_chunks):
    nw = idx.shape[0]
    b_per_w = n_chunks * CHUNK
    n_outer = n_chunks // NBUF
    assert n_outer * NBUF == n_chunks
    mesh = plsc.VectorSubcoreMesh(core_axis_name="c", subcore_axis_name="s")

    @functools.partial(
        pl.kernel,
        mesh=mesh,
        out_type=jax.ShapeDtypeStruct((nw * b_per_w, D), jnp.float32),
        scratch_types=(
            [pltpu.VMEM((n_chunks, CHUNK), jnp.int32),
             pltpu.VMEM((NBUF, CHUNK, D), jnp.float32)]
            + [pltpu.SemaphoreType.DMA] * (2 * NBUF)
        ),
    )
    def k(table_hbm, idx_hbm, out_hbm, idx_v, rows_v, *sems):
        gsem = sems[:NBUF]
        osem = sems[NBUF:]
        wid = lax.axis_index("s") * 2 + lax.axis_index("c")
        base = wid * b_per_w
        pltpu.sync_copy(idx_hbm.at[wid], idx_v)

        def fire(h, slot):
            pltpu.async_copy(
                table_hbm.at[idx_v.at[h]], rows_v.at[slot], gsem[slot])

        def wait_gather(g, slot):
            pltpu.make_async_copy(
                table_hbm.at[idx_v.at[g]], rows_v.at[slot], gsem[slot]).wait()

        def out_copy(slot, g):
            return pltpu.make_async_copy(
                rows_v.at[slot],
                out_hbm.at[pl.ds(base + g * CHUNK, CHUNK)],
                osem[slot])

        for h in range(AHEAD):
            fire(h, h)

        def body(t, carry):
            for b in range(NBUF):
                g = t * NBUF + b
                sh = (b + AHEAD) % NBUF
                h = g + AHEAD

                @pl.when(h < n_chunks)
                def _():
                    @pl.when(h >= NBUF)
                    def _():
                        out_copy(sh, 0).wait()
                    fire(h, sh)

                wait_gather(g, b)
                out_copy(b, g).start()
            return carry

        lax.fori_loop(0, n_outer, body, 0)

        for b in range(NBUF):
            out_copy(b, 0).wait()

    return k(table, idx)


NREP = 16


def kernel(indices, embed_weight):
    bsz, seq = indices.shape
    total = bsz * seq
    nw = 32
    n_chunks = total // (nw * CHUNK)
    v = embed_weight.shape[0]
    table_rep = jnp.tile(embed_weight, (NREP, 1))
    idx = indices.reshape(nw, n_chunks, CHUNK).astype(jnp.int32)
    rep_off = (jnp.arange(nw, dtype=jnp.int32) % NREP * v).reshape(nw, 1, 1)
    out = _sc_gather(table_rep, idx + rep_off, n_chunks)
    return out.reshape(bsz, seq, D)


# speedup vs baseline: 1.7524x; 1.0286x over previous
import functools

import jax
import jax.numpy as jnp
from jax import lax
from jax.experimental import pallas as pl
from jax.experimental.pallas import tpu as pltpu
from jax.experimental.pallas import tpu_sc as plsc

D = 128
SEQ = 50
SEQ_PAD = 56
ENT_PER = 2
NBUF = 4
AHEAD = 2
NW = 32
NREP = 16


@functools.partial(jax.jit, static_argnames=("ent_per_w",))
def _sc_gather(table, idx, ent_per_w):
    n_streams = ent_per_w // ENT_PER
    n_outer = n_streams // NBUF
    assert n_outer * NBUF == n_streams
    idx_per_w = ent_per_w * SEQ_PAD
    mesh = plsc.VectorSubcoreMesh(core_axis_name="c", subcore_axis_name="s")

    @functools.partial(
        pl.kernel,
        mesh=mesh,
        out_type=jax.ShapeDtypeStruct((NW * ent_per_w, SEQ, D), jnp.float32),
        scratch_types=(
            [pltpu.VMEM((idx_per_w,), jnp.int32),
             pltpu.VMEM((NBUF, ENT_PER * SEQ_PAD, D), jnp.float32)]
            + [pltpu.SemaphoreType.DMA] * (2 * NBUF)
        ),
        compiler_params=pltpu.CompilerParams(use_tc_tiling_on_sc=True),
    )
    def k(table_hbm, idx_hbm, out_hbm, idx_v, rows_v, *sems):
        gsem = sems[:NBUF]
        osem = sems[NBUF:]
        wid = lax.axis_index("s") * 2 + lax.axis_index("c")
        e_base = wid * ent_per_w
        pltpu.sync_copy(idx_hbm.at[wid], idx_v)

        def gather(s, slot):
            return pltpu.make_async_copy(
                table_hbm.at[idx_v.at[pl.ds(s * (ENT_PER * SEQ_PAD),
                                            ENT_PER * SEQ_PAD)]],
                rows_v.at[slot], gsem[slot])

        def out_copy(slot, j, s):
            return pltpu.make_async_copy(
                rows_v.at[slot].at[pl.ds(j * SEQ_PAD, SEQ)],
                out_hbm.at[e_base + s * ENT_PER + j],
                osem[slot])

        for h in range(AHEAD):
            gather(h, h).start()

        def body(t, carry):
            for b in range(NBUF):
                s = t * NBUF + b
                sh = (b + AHEAD) % NBUF
                h = s + AHEAD

                @pl.when(h < n_streams)
                def _():
                    @pl.when(h >= NBUF)
                    def _():
                        for j in range(ENT_PER):
                            out_copy(sh, j, 0).wait()
                    gather(h, sh).start()

                gather(s, b).wait()
                for j in range(ENT_PER):
                    out_copy(b, j, s).start()
            return carry

        lax.fori_loop(0, n_outer, body, 0)

        for b in range(NBUF):
            for j in range(ENT_PER):
                out_copy(b, j, 0).wait()

    return k(table, idx)


def kernel(indices, embed_weight):
    bsz, seq = indices.shape
    v = embed_weight.shape[0]
    ent_per_w = bsz // NW
    table_rep = jnp.tile(embed_weight, (NREP, 1))
    idx_p = jnp.pad(indices.astype(jnp.int32), ((0, 0), (0, SEQ_PAD - seq)))
    idx_w = idx_p.reshape(NW, ent_per_w * SEQ_PAD)
    rep_off = (jnp.arange(NW, dtype=jnp.int32) % NREP * v)[:, None]
    return _sc_gather(table_rep, idx_w + rep_off, ent_per_w)
